# Initial kernel scaffold; baseline (speedup 1.0000x reference)
import jax
import jax.numpy as jnp
from jax.experimental import pallas as pl


def kernel(z, embedding):
    raise NotImplementedError("write your pallas kernel here")


# speedup vs baseline: 1.7289x; 1.7289x over previous
import jax
import jax.numpy as jnp
from jax.experimental import pallas as pl
from jax.experimental.pallas import tpu as pltpu

N_CODES = 1024
DIM = 256
HW = 1024
B = 16
BETA = 0.25


def _vq_body(zp_ref, emb_ref, out_ref, idx_ref, loss_ref):
    b = pl.program_id(0)
    zp = zp_ref[0]
    emb = emb_ref[...]
    mm = jax.lax.dot_general(zp, emb, (((1,), (1,)), ((), ())),
                             preferred_element_type=jnp.float32)
    z2 = jnp.sum(zp * zp, axis=1, keepdims=True)
    e2 = jnp.sum(emb * emb, axis=1)
    d = (z2 + e2[None, :]) - 2.0 * mm
    m = jnp.min(d, axis=1, keepdims=True)
    col = jax.lax.broadcasted_iota(jnp.int32, d.shape, 1)
    idx = jnp.min(jnp.where(d == m, col, jnp.int32(2**30)), axis=1)
    idx_ref[0, 0, :] = idx
    onehot = (jax.lax.broadcasted_iota(jnp.int32, (N_CODES, HW), 0)
              == idx[None, :]).astype(jnp.float32)
    out_ref[0] = jax.lax.dot_general(emb, onehot, (((0,), (0,)), ((), ())),
                                     preferred_element_type=jnp.float32)

    @pl.when(b == 0)
    def _init():
        loss_ref[...] = jnp.zeros((1, 1), jnp.float32)

    loss_ref[...] += jnp.sum(m).reshape(1, 1)


def kernel(z, embedding):
    zp = jnp.transpose(z, (0, 2, 3, 1)).reshape(B, HW, DIM)
    out3, idx3, loss11 = pl.pallas_call(
        _vq_body,
        grid=(B,),
        in_specs=[
            pl.BlockSpec((1, HW, DIM), lambda b: (b, 0, 0)),
            pl.BlockSpec((N_CODES, DIM), lambda b: (0, 0)),
        ],
        out_specs=[
            pl.BlockSpec((1, DIM, HW), lambda b: (b, 0, 0)),
            pl.BlockSpec((1, 1, HW), lambda b: (b, 0, 0)),
            pl.BlockSpec((1, 1), lambda b: (0, 0)),
        ],
        out_shape=[
            jax.ShapeDtypeStruct((B, DIM, HW), jnp.float32),
            jax.ShapeDtypeStruct((B, 1, HW), jnp.int32),
            jax.ShapeDtypeStruct((1, 1), jnp.float32),
        ],
    )(zp, embedding)
    out = out3.reshape(z.shape)
    idx = idx3.reshape(-1)
    loss = loss11[0, 0] * (BETA / (B * HW * DIM))
    return out, loss, idx
